# baseline (speedup 1.0000x reference)
import functools

import jax
import jax.numpy as jnp
from jax import lax
from jax.experimental import pallas as pl
from jax.experimental.pallas import tpu as pltpu
from jax.experimental.pallas import tpu_sc as plsc

B = 1024
D = 256
N = 16384
IMG = 4096
TILE = 2048
NT = N // TILE

NC, NS = 2, 16
NW = NC * NS
B_PER_W = B // NW
CHUNK = 16
N_CHUNKS = B_PER_W // CHUNK


def _topk_body(q_ref, f_ref, scores_ref, idx_ref, qn_ref):
    i = pl.program_id(0)

    @pl.when(i == 0)
    def _():
        q = q_ref[...]
        n = jnp.sqrt(jnp.sum(q * q, axis=1, keepdims=True))
        qn_ref[...] = q / jnp.clip(n, 1e-12, None)

    sim = lax.dot_general(
        qn_ref[...], f_ref[...],
        dimension_numbers=(((1,), (1,)), ((), ())),
        preferred_element_type=jnp.float32,
    )
    m = jnp.max(sim, axis=1, keepdims=True)
    pos = lax.broadcasted_iota(jnp.int32, (B, TILE), 1)
    a = jnp.min(jnp.where(sim == m, pos, TILE), axis=1, keepdims=True) + i * TILE

    @pl.when(i == 0)
    def _():
        scores_ref[...] = m
        idx_ref[...] = a

    @pl.when(i > 0)
    def _():
        prev = scores_ref[...]
        better = m > prev
        scores_ref[...] = jnp.where(better, m, prev)
        idx_ref[...] = jnp.where(better, a, idx_ref[...])


def _topk(q, features):
    return pl.pallas_call(
        _topk_body,
        grid=(NT,),
        in_specs=[
            pl.BlockSpec((B, D), lambda i: (0, 0)),
            pl.BlockSpec((TILE, D), lambda i: (i, 0)),
        ],
        out_specs=[
            pl.BlockSpec((B, 1), lambda i: (0, 0)),
            pl.BlockSpec((B, 1), lambda i: (0, 0)),
        ],
        out_shape=[
            jax.ShapeDtypeStruct((B, 1), jnp.float32),
            jax.ShapeDtypeStruct((B, 1), jnp.int32),
        ],
        scratch_shapes=[pltpu.VMEM((B, D), jnp.float32)],
    )(q, features)


def _gather_body(img_hbm, idx_hbm, out_hbm, idx_v, rows_v, sem):
    wid = lax.axis_index("s") * NC + lax.axis_index("c")
    base = wid * B_PER_W
    for c in range(N_CHUNKS):
        off = base + c * CHUNK
        pltpu.sync_copy(idx_hbm.at[pl.ds(off, CHUNK)], idx_v)
        pltpu.async_copy(img_hbm.at[idx_v], rows_v, sem).wait()
        pltpu.sync_copy(rows_v, out_hbm.at[pl.ds(off, CHUNK)])


def _gather(img_flat, idx):
    mesh = plsc.VectorSubcoreMesh(core_axis_name="c", subcore_axis_name="s")
    k = pl.kernel(
        _gather_body,
        out_type=jax.ShapeDtypeStruct((B, IMG), jnp.float32),
        mesh=mesh,
        scratch_types=[
            pltpu.VMEM((CHUNK,), jnp.int32),
            pltpu.VMEM((CHUNK, IMG), jnp.float32),
            pltpu.SemaphoreType.DMA,
        ],
    )
    return k(img_flat, idx)


def kernel(query_features, features, images):
    scores2, idx2 = _topk(query_features, features)
    idx = idx2.reshape(B)
    img_flat = images.reshape(N, IMG)
    out = _gather(img_flat, idx)
    return out.reshape(B, 1, 64, 64), scores2.reshape(B)
